# baseline (speedup 1.0000x reference)
import functools
import math

import numpy as np
import jax
import jax.numpy as jnp
from jax import lax
from jax.experimental import pallas as pl
from jax.experimental.pallas import tpu as pltpu
from jax.experimental.pallas import tpu_sc as plsc

_B, _N, _V, _D = 16, 4096, 21, 256
_L = 16
_NW = 32
_CHUNK = _N // _NW


@functools.lru_cache(maxsize=None)
def _pos_table():
    pos = np.arange(_N, dtype=np.float32)[:, None]
    div = np.exp(
        np.arange(0, _D, 2, dtype=np.float32) * (-math.log(10000.0) / _D)
    ).astype(np.float32)
    enc = np.zeros((_N, _D), dtype=np.float32)
    enc[:, 0::2] = np.sin(pos * div)
    enc[:, 1::2] = np.cos(pos * div)
    return enc


_mesh = plsc.VectorSubcoreMesh(core_axis_name="c", subcore_axis_name="s")


@functools.partial(
    pl.kernel,
    out_type=jax.ShapeDtypeStruct((_B * _N * _D,), jnp.float32),
    mesh=_mesh,
    compiler_params=pltpu.CompilerParams(needs_layout_passes=False),
    scratch_types=[
        pltpu.VMEM((_V * _D,), jnp.float32),
        pltpu.VMEM((_CHUNK * _D,), jnp.float32),
        pltpu.VMEM((_B * _CHUNK,), jnp.int32),
        pltpu.VMEM((_D,), jnp.float32),
        pltpu.VMEM((_D,), jnp.float32),
        pltpu.VMEM((_CHUNK * _D,), jnp.float32),
    ],
)
def _sc_embed_ln(tokens_hbm, table_hbm, gamma_hbm, beta_hbm, pos_hbm,
                 out_hbm, table_v, pos_v, tok_v, gamma_v, beta_v, out_v):
    wid = lax.axis_index("s") * 2 + lax.axis_index("c")
    n0 = wid * _CHUNK

    pltpu.sync_copy(table_hbm, table_v)
    pltpu.sync_copy(pos_hbm.at[pl.ds(n0 * _D, _CHUNK * _D)], pos_v)
    pltpu.sync_copy(gamma_hbm, gamma_v)
    pltpu.sync_copy(beta_hbm, beta_v)
    for b in range(_B):
        pltpu.sync_copy(tokens_hbm.at[pl.ds(b * _N + n0, _CHUNK)],
                        tok_v.at[pl.ds(b * _CHUNK, _CHUNK)])

    iota = lax.iota(jnp.int32, _L)
    inv_d = jnp.float32(1.0 / _D)

    def batch_body(b, carry):
        def row_body(r, rcarry):
            tokv = plsc.load_gather(tok_v, [jnp.full((_L,), b * _CHUNK + r,
                                                     jnp.int32)])
            tbase = tokv * _D
            pbase = jnp.full((_L,), r * _D, jnp.int32) + iota
            xs = []
            s = jnp.zeros((_L,), jnp.float32)
            q = jnp.zeros((_L,), jnp.float32)
            for j in range(_D // _L):
                t = plsc.load_gather(table_v, [tbase + (iota + j * _L)])
                p = plsc.load_gather(pos_v, [pbase + (j * _L)])
                x = t + p
                xs.append(x)
                s = s + x
                q = q + x * x
            mean = jnp.sum(s) * inv_d
            var = jnp.sum(q) * inv_d - mean * mean
            vv = jnp.full((_L,), var + jnp.float32(1e-5), jnp.float32)
            yi = jnp.int32(0x5F3759DF) - lax.shift_right_logical(
                plsc.bitcast(vv, jnp.int32), 1)
            y = plsc.bitcast(yi, jnp.float32)
            for _ in range(3):
                y = y * (jnp.float32(1.5) - jnp.float32(0.5) * vv * y * y)
            meanv = jnp.full((_L,), mean, jnp.float32)
            for j in range(_D // _L):
                g = gamma_v[pl.ds(j * _L, _L)]
                bb = beta_v[pl.ds(j * _L, _L)]
                o = (xs[j] - meanv) * y * g + bb
                plsc.store_scatter(out_v, [pbase + (j * _L)], o)
            return rcarry

        lax.fori_loop(0, _CHUNK, row_body, 0)
        pltpu.sync_copy(out_v,
                        out_hbm.at[pl.ds((b * _N + n0) * _D, _CHUNK * _D)])
        return carry

    lax.fori_loop(0, _B, batch_body, 0)


def kernel(tokens, table, gamma, beta):
    tokens = tokens.astype(jnp.int32).reshape(-1)
    pos = jnp.asarray(_pos_table()).reshape(-1)
    out = _sc_embed_ln(tokens, table.reshape(-1), gamma, beta, pos)
    return out.reshape(_B, _N, _D)


# speedup vs baseline: 1.4716x; 1.4716x over previous
import functools
import math

import numpy as np
import jax
import jax.numpy as jnp
from jax import lax
from jax.experimental import pallas as pl
from jax.experimental.pallas import tpu as pltpu
from jax.experimental.pallas import tpu_sc as plsc

_B, _N, _V, _D = 16, 4096, 21, 256
_VP = 32
_L = 16
_NW = 32
_CHUNK = _N // _NW
_TCB = 512


@functools.lru_cache(maxsize=None)
def _pos_table():
    pos = np.arange(_N, dtype=np.float32)[:, None]
    div = np.exp(
        np.arange(0, _D, 2, dtype=np.float32) * (-math.log(10000.0) / _D)
    ).astype(np.float32)
    enc = np.zeros((_N, _D), dtype=np.float32)
    enc[:, 0::2] = np.sin(pos * div)
    enc[:, 1::2] = np.cos(pos * div)
    return enc


def _tc_stats_body(pos_ref, tbl_ref, mean_ref, rstd_ref):
    inv_d = jnp.float32(1.0 / _D)
    pos_b = pos_ref[...]
    tbl = tbl_ref[...]
    c = lax.dot_general(pos_b, tbl, (((1,), (1,)), ((), ())),
                        preferred_element_type=jnp.float32)
    s1 = (jnp.sum(tbl, axis=1) * inv_d)[None, :]
    s2 = (jnp.sum(tbl * tbl, axis=1) * inv_d)[None, :]
    p1 = jnp.sum(pos_b, axis=1, keepdims=True) * inv_d
    p2 = jnp.sum(pos_b * pos_b, axis=1, keepdims=True) * inv_d
    mean = p1 + s1
    var = s2 + (jnp.float32(2.0) * inv_d) * c + p2 - mean * mean
    mean_ref[...] = mean + jnp.zeros_like(c)
    rstd_ref[...] = lax.rsqrt(var + jnp.float32(1e-5))


_tc_stats = pl.pallas_call(
    _tc_stats_body,
    grid=(_N // _TCB,),
    in_specs=[
        pl.BlockSpec((_TCB, _D), lambda i: (i, 0)),
        pl.BlockSpec((_VP, _D), lambda i: (0, 0)),
    ],
    out_specs=[
        pl.BlockSpec((_TCB, _VP), lambda i: (i, 0)),
        pl.BlockSpec((_TCB, _VP), lambda i: (i, 0)),
    ],
    out_shape=[
        jax.ShapeDtypeStruct((_N, _VP), jnp.float32),
        jax.ShapeDtypeStruct((_N, _VP), jnp.float32),
    ],
)


_mesh = plsc.VectorSubcoreMesh(core_axis_name="c", subcore_axis_name="s")


@functools.partial(
    pl.kernel,
    out_type=jax.ShapeDtypeStruct((_B * _N * _D,), jnp.float32),
    mesh=_mesh,
    compiler_params=pltpu.CompilerParams(needs_layout_passes=False),
    scratch_types=[
        pltpu.VMEM((_V * _D,), jnp.float32),
        pltpu.VMEM((_CHUNK * _D,), jnp.float32),
        pltpu.VMEM((_B * _CHUNK,), jnp.int32),
        pltpu.VMEM((_D,), jnp.float32),
        pltpu.VMEM((_D,), jnp.float32),
        pltpu.VMEM((_CHUNK * _VP,), jnp.float32),
        pltpu.VMEM((_CHUNK * _VP,), jnp.float32),
        pltpu.VMEM((_CHUNK * _D,), jnp.float32),
    ],
)
def _sc_embed_ln(tokens_hbm, table_hbm, gamma_hbm, beta_hbm, pos_hbm,
                 mean_hbm, rstd_hbm, out_hbm,
                 table_v, pos_v, tok_v, gamma_v, beta_v, mean_v, rstd_v,
                 out_v):
    wid = lax.axis_index("s") * 2 + lax.axis_index("c")
    n0 = wid * _CHUNK

    pltpu.sync_copy(table_hbm, table_v)
    pltpu.sync_copy(pos_hbm.at[pl.ds(n0 * _D, _CHUNK * _D)], pos_v)
    pltpu.sync_copy(gamma_hbm, gamma_v)
    pltpu.sync_copy(beta_hbm, beta_v)
    pltpu.sync_copy(mean_hbm.at[pl.ds(n0 * _VP, _CHUNK * _VP)], mean_v)
    pltpu.sync_copy(rstd_hbm.at[pl.ds(n0 * _VP, _CHUNK * _VP)], rstd_v)
    for b in range(_B):
        pltpu.sync_copy(tokens_hbm.at[pl.ds(b * _N + n0, _CHUNK)],
                        tok_v.at[pl.ds(b * _CHUNK, _CHUNK)])

    iota = lax.iota(jnp.int32, _L)

    def batch_body(b, carry):
        @plsc.parallel_loop(0, _CHUNK, step=1, unroll=2)
        def row_body(r):
            tokv = plsc.load_gather(
                tok_v, [jnp.full((_L,), b * _CHUNK + r, jnp.int32)])
            sidx = tokv + r * _VP
            m = plsc.load_gather(mean_v, [sidx])
            rs = plsc.load_gather(rstd_v, [sidx])
            tbase = tokv * _D
            pbase = iota + r * _D
            for j in range(_D // _L):
                t = plsc.load_gather(table_v, [tbase + (iota + j * _L)])
                p = plsc.load_gather(pos_v, [pbase + (j * _L)])
                g = gamma_v[pl.ds(j * _L, _L)]
                bb = beta_v[pl.ds(j * _L, _L)]
                o = (t + p - m) * rs * g + bb
                plsc.store_scatter(out_v, [pbase + (j * _L)], o)

        pltpu.sync_copy(out_v,
                        out_hbm.at[pl.ds((b * _N + n0) * _D, _CHUNK * _D)])
        return carry

    lax.fori_loop(0, _B, batch_body, 0)


def kernel(tokens, table, gamma, beta):
    tokens = tokens.astype(jnp.int32).reshape(-1)
    pos = jnp.asarray(_pos_table())
    tbl_pad = jnp.pad(table, ((0, _VP - _V), (0, 0)))
    mean_arr, rstd_arr = _tc_stats(pos, tbl_pad)
    out = _sc_embed_ln(tokens, table.reshape(-1), gamma, beta,
                       pos.reshape(-1), mean_arr.reshape(-1),
                       rstd_arr.reshape(-1))
    return out.reshape(_B, _N, _D)


# speedup vs baseline: 2.5119x; 1.7070x over previous
import functools
import math

import numpy as np
import jax
import jax.numpy as jnp
from jax import lax
from jax.experimental import pallas as pl
from jax.experimental.pallas import tpu as pltpu
from jax.experimental.pallas import tpu_sc as plsc

_B, _N, _V, _D = 16, 4096, 21, 256
_VP = 32
_L = 16
_NW = 32
_CHUNK = _N // _NW
_HB = 64
_NG = _B * _CHUNK // _HB


@functools.lru_cache(maxsize=None)
def _pos_table():
    pos = np.arange(_N, dtype=np.float32)[:, None]
    div = np.exp(
        np.arange(0, _D, 2, dtype=np.float32) * (-math.log(10000.0) / _D)
    ).astype(np.float32)
    enc = np.zeros((_N, _D), dtype=np.float32)
    enc[:, 0::2] = np.sin(pos * div)
    enc[:, 1::2] = np.cos(pos * div)
    return enc


def _tc_stats_body(pos_ref, tbl_ref, p1_ref, p2_ref, mean_ref, rstd_ref):
    inv_d = jnp.float32(1.0 / _D)
    tbl = tbl_ref[...]
    c = lax.dot_general(pos_ref[...], tbl, (((1,), (1,)), ((), ())),
                        preferred_element_type=jnp.float32)
    s1 = (jnp.sum(tbl, axis=1) * inv_d)[None, :]
    s2 = (jnp.sum(tbl * tbl, axis=1) * inv_d)[None, :]
    mean = p1_ref[...] + s1
    var = s2 + (jnp.float32(2.0) * inv_d) * c + p2_ref[...] - mean * mean
    mean_ref[...] = mean
    rstd_ref[...] = lax.rsqrt(var + jnp.float32(1e-5))


_TCB = 512

_tc_stats = pl.pallas_call(
    _tc_stats_body,
    grid=(_N // _TCB,),
    in_specs=[
        pl.BlockSpec((_TCB, _D), lambda i: (i, 0)),
        pl.BlockSpec((_VP, _D), lambda i: (0, 0)),
        pl.BlockSpec((_TCB, 1), lambda i: (i, 0)),
        pl.BlockSpec((_TCB, 1), lambda i: (i, 0)),
    ],
    out_specs=[
        pl.BlockSpec((_TCB, _VP), lambda i: (i, 0)),
        pl.BlockSpec((_TCB, _VP), lambda i: (i, 0)),
    ],
    out_shape=[
        jax.ShapeDtypeStruct((_N, _VP), jnp.float32),
        jax.ShapeDtypeStruct((_N, _VP), jnp.float32),
    ],
)


_mesh = plsc.VectorSubcoreMesh(core_axis_name="c", subcore_axis_name="s")


@functools.partial(
    pl.kernel,
    out_type=jax.ShapeDtypeStruct((_B, _N, _D), jnp.float32),
    mesh=_mesh,
    compiler_params=pltpu.CompilerParams(needs_layout_passes=False),
    scratch_types=[
        pltpu.VMEM((_V, _D), jnp.float32),
        pltpu.VMEM((_CHUNK, _D), jnp.float32),
        pltpu.VMEM((_B, _CHUNK), jnp.int32),
        pltpu.VMEM((_D,), jnp.float32),
        pltpu.VMEM((_D,), jnp.float32),
        pltpu.VMEM((_CHUNK, _VP), jnp.float32),
        pltpu.VMEM((_CHUNK, _VP), jnp.float32),
        pltpu.VMEM((2, _HB, _D), jnp.float32),
        pltpu.SemaphoreType.DMA((2,)),
    ],
)
def _sc_embed_ln(tokens_hbm, table_hbm, gamma_hbm, beta_hbm, pos_hbm,
                 mean_hbm, rstd_hbm, out_hbm,
                 table_v, pos_v, tok_v, gamma_v, beta_v, mean_v, rstd_v,
                 out_v, sem):
    wid = lax.axis_index("s") * 2 + lax.axis_index("c")
    n0 = wid * _CHUNK

    pltpu.sync_copy(table_hbm, table_v)
    pltpu.sync_copy(pos_hbm.at[pl.ds(n0, _CHUNK)], pos_v)
    pltpu.sync_copy(tokens_hbm.at[:, pl.ds(n0, _CHUNK)], tok_v)
    pltpu.sync_copy(gamma_hbm, gamma_v)
    pltpu.sync_copy(beta_hbm, beta_v)
    pltpu.sync_copy(mean_hbm.at[pl.ds(n0, _CHUNK)], mean_v)
    pltpu.sync_copy(rstd_hbm.at[pl.ds(n0, _CHUNK)], rstd_v)

    gpb = _CHUNK // _HB

    def group_body(gi, carry):
        b = gi // gpb
        g = gi % gpb
        par = gi % 2
        r0 = g * _HB

        @pl.when(gi >= 2)
        def _wait_prev():
            pltpu.make_async_copy(
                out_v.at[par],
                out_hbm.at[b, pl.ds(n0 + r0, _HB)],
                sem.at[par],
            ).wait()

        @plsc.parallel_loop(0, _HB, step=1, unroll=2)
        def row_body(rl):
            r = r0 + rl
            tokv = plsc.load_gather(
                tok_v, [jnp.full((_L,), b, jnp.int32),
                        jnp.full((_L,), r, jnp.int32)])
            rfull = jnp.full((_L,), r, jnp.int32)
            m = plsc.load_gather(mean_v, [rfull, tokv])
            rs = plsc.load_gather(rstd_v, [rfull, tokv])
            iota = lax.iota(jnp.int32, _L)
            for j in range(_D // _L):
                t = plsc.load_gather(table_v, [tokv, iota + (j * _L)])
                p = pos_v[r, pl.ds(j * _L, _L)]
                g_ = gamma_v[pl.ds(j * _L, _L)]
                b_ = beta_v[pl.ds(j * _L, _L)]
                out_v[par, rl, pl.ds(j * _L, _L)] = (t + p - m) * rs * g_ + b_

        pltpu.async_copy(
            out_v.at[par],
            out_hbm.at[b, pl.ds(n0 + r0, _HB)],
            sem.at[par],
        )
        return carry

    lax.fori_loop(0, _NG, group_body, 0)

    for par in range(2):
        pltpu.make_async_copy(
            out_v.at[par],
            out_hbm.at[0, pl.ds(n0, _HB)],
            sem.at[par],
        ).wait()


def kernel(tokens, table, gamma, beta):
    tokens = tokens.astype(jnp.int32)
    posn = _pos_table()
    pos = jnp.asarray(posn)
    p1 = jnp.asarray(posn.mean(axis=1, keepdims=True))
    p2 = jnp.asarray((posn * posn).mean(axis=1, keepdims=True))
    tbl_pad = jnp.pad(table, ((0, _VP - _V), (0, 0)))
    mean_arr, rstd_arr = _tc_stats(pos, tbl_pad, p1, p2)
    return _sc_embed_ln(tokens, table, gamma, beta, pos, mean_arr, rstd_arr)


# speedup vs baseline: 4.4034x; 1.7530x over previous
import functools
import math

import numpy as np
import jax
import jax.numpy as jnp
from jax import lax
from jax.experimental import pallas as pl
from jax.experimental.pallas import tpu as pltpu
from jax.experimental.pallas import tpu_sc as plsc

_B, _N, _V, _D = 16, 4096, 21, 256
_VP = 32
_L = 16
_NW = 32
_CHUNK = _N // _NW
_HB = 64
_NG = _B * _CHUNK // _HB
_TCB = 512


@functools.lru_cache(maxsize=None)
def _pos_table():
    pos = np.arange(_N, dtype=np.float32)[:, None]
    div = np.exp(
        np.arange(0, _D, 2, dtype=np.float32) * (-math.log(10000.0) / _D)
    ).astype(np.float32)
    enc = np.zeros((_N, _D), dtype=np.float32)
    enc[:, 0::2] = np.sin(pos * div)
    enc[:, 1::2] = np.cos(pos * div)
    return enc


def _tc_vocab_body(tbl_ref, gamma_ref, sv_ref, ta_ref):
    inv_d = jnp.float32(1.0 / _D)
    tbl = tbl_ref[...]
    s1 = jnp.sum(tbl, axis=1) * inv_d
    s2 = jnp.sum(tbl * tbl, axis=1) * inv_d
    sv_ref[...] = jnp.stack(
        [s1, s2, s1, s2, s1, s2, s1, s2], axis=0)
    ta_ref[...] = gamma_ref[...] * (tbl - s1[:, None])


_tc_vocab = pl.pallas_call(
    _tc_vocab_body,
    out_shape=[
        jax.ShapeDtypeStruct((8, _VP), jnp.float32),
        jax.ShapeDtypeStruct((_VP, _D), jnp.float32),
    ],
)


def _tc_stats_body(pos_ref, tbl_ref, sv_ref, p1_ref, p2_ref, gamma_ref,
                   rstd_ref, pa_ref):
    inv_d = jnp.float32(1.0 / _D)
    pos_b = pos_ref[...]
    c = lax.dot_general(pos_b, tbl_ref[...], (((1,), (1,)), ((), ())),
                        preferred_element_type=jnp.float32)
    s1 = sv_ref[0, :][None, :]
    s2 = sv_ref[1, :][None, :]
    p1 = p1_ref[...]
    mean = p1 + s1
    var = s2 + (jnp.float32(2.0) * inv_d) * c + p2_ref[...] - mean * mean
    rstd_ref[...] = lax.rsqrt(var + jnp.float32(1e-5))
    pa_ref[...] = gamma_ref[...] * (pos_b - p1)


_tc_stats = pl.pallas_call(
    _tc_stats_body,
    grid=(_N // _TCB,),
    in_specs=[
        pl.BlockSpec((_TCB, _D), lambda i: (i, 0)),
        pl.BlockSpec((_VP, _D), lambda i: (0, 0)),
        pl.BlockSpec((8, _VP), lambda i: (0, 0)),
        pl.BlockSpec((_TCB, 1), lambda i: (i, 0)),
        pl.BlockSpec((_TCB, 1), lambda i: (i, 0)),
        pl.BlockSpec((1, _D), lambda i: (0, 0)),
    ],
    out_specs=[
        pl.BlockSpec((_TCB, _VP), lambda i: (i, 0)),
        pl.BlockSpec((_TCB, _D), lambda i: (i, 0)),
    ],
    out_shape=[
        jax.ShapeDtypeStruct((_N, _VP), jnp.float32),
        jax.ShapeDtypeStruct((_N, _D), jnp.float32),
    ],
)


_mesh = plsc.VectorSubcoreMesh(core_axis_name="c", subcore_axis_name="s")


@functools.partial(
    pl.kernel,
    out_type=jax.ShapeDtypeStruct((_B, _N, _D), jnp.float32),
    mesh=_mesh,
    compiler_params=pltpu.CompilerParams(needs_layout_passes=False),
    scratch_types=[
        pltpu.VMEM((_VP, _D), jnp.float32),
        pltpu.VMEM((_CHUNK, _D), jnp.float32),
        pltpu.VMEM((_B, _CHUNK), jnp.int32),
        pltpu.VMEM((_D,), jnp.float32),
        pltpu.VMEM((_CHUNK, _VP), jnp.float32),
        pltpu.VMEM((2, _HB, _D), jnp.float32),
        pltpu.SemaphoreType.DMA((2,)),
        pltpu.SemaphoreType.DMA,
    ],
)
def _sc_embed_ln(tokens_hbm, ta_hbm, beta_hbm, pa_hbm, rstd_hbm, out_hbm,
                 ta_v, pa_v, tok_v, beta_v, rstd_v, out_v, sem, ssem):
    wid = lax.axis_index("s") * 2 + lax.axis_index("c")
    n0 = wid * _CHUNK

    copies = [
        pltpu.async_copy(ta_hbm, ta_v, ssem),
        pltpu.async_copy(pa_hbm.at[pl.ds(n0, _CHUNK)], pa_v, ssem),
        pltpu.async_copy(tokens_hbm.at[:, pl.ds(n0, _CHUNK)], tok_v, ssem),
        pltpu.async_copy(beta_hbm, beta_v, ssem),
        pltpu.async_copy(rstd_hbm.at[pl.ds(n0, _CHUNK)], rstd_v, ssem),
    ]
    for cp in copies:
        cp.wait()

    bchunks = [beta_v[pl.ds(j * _L, _L)] for j in range(_D // _L)]

    gpb = _CHUNK // _HB

    def group_body(gi, carry):
        b = gi // gpb
        g = gi % gpb
        par = gi % 2
        r0 = g * _HB

        @pl.when(gi >= 2)
        def _wait_prev():
            pltpu.make_async_copy(
                out_v.at[par],
                out_hbm.at[b, pl.ds(n0 + r0, _HB)],
                sem.at[par],
            ).wait()

        @plsc.parallel_loop(0, _HB, step=1, unroll=2)
        def row_body(rl):
            r = r0 + rl
            tokv = plsc.load_gather(
                tok_v, [jnp.full((_L,), b, jnp.int32),
                        jnp.full((_L,), r, jnp.int32)])
            rs = plsc.load_gather(
                rstd_v, [jnp.full((_L,), r, jnp.int32), tokv])
            iota = lax.iota(jnp.int32, _L)
            for j in range(_D // _L):
                t = plsc.load_gather(ta_v, [tokv, iota + (j * _L)])
                p = pa_v[r, pl.ds(j * _L, _L)]
                out_v[par, rl, pl.ds(j * _L, _L)] = (
                    (t + p) * rs + bchunks[j])

        pltpu.async_copy(
            out_v.at[par],
            out_hbm.at[b, pl.ds(n0 + r0, _HB)],
            sem.at[par],
        )
        return carry

    lax.fori_loop(0, _NG, group_body, 0)

    for par in range(2):
        pltpu.make_async_copy(
            out_v.at[par],
            out_hbm.at[0, pl.ds(n0, _HB)],
            sem.at[par],
        ).wait()


def kernel(tokens, table, gamma, beta):
    tokens = tokens.astype(jnp.int32)
    posn = _pos_table()
    pos = jnp.asarray(posn)
    p1 = jnp.asarray(posn.mean(axis=1, keepdims=True))
    p2 = jnp.asarray((posn * posn).mean(axis=1, keepdims=True))
    tbl_pad = jnp.pad(table, ((0, _VP - _V), (0, 0)))
    gamma2d = gamma[None, :]
    sv, ta = _tc_vocab(tbl_pad, gamma2d)
    rstd_arr, pa = _tc_stats(pos, tbl_pad, sv, p1, p2, gamma2d)
    return _sc_embed_ln(tokens, ta, beta, pa, rstd_arr)
